# baseline (speedup 1.0000x reference)
import functools

import jax
import jax.numpy as jnp
from jax import lax
from jax.experimental import pallas as pl
from jax.experimental.pallas import tpu as pltpu
from jax.experimental.pallas import tpu_sc as plsc

_N_TABLE = 128
_LANES = 16


def _sc_lookup(m_total, n_opt, n_cores, chunk):
    mesh = plsc.VectorSubcoreMesh(
        core_axis_name="c", subcore_axis_name="s", num_cores=n_cores
    )

    offs = list(range(0, n_opt - _LANES + 1, _LANES))
    if n_opt % _LANES:
        offs.append(n_opt - _LANES)

    @functools.partial(
        pl.kernel,
        out_type=jax.ShapeDtypeStruct((m_total,), jnp.float32),
        mesh=mesh,
        compiler_params=pltpu.CompilerParams(needs_layout_passes=False),
        scratch_types=[
            pltpu.VMEM((n_opt,), jnp.float32),
            pltpu.VMEM((n_opt,), jnp.int32),
            pltpu.VMEM((_N_TABLE,), jnp.float32),
            pltpu.VMEM((chunk,), jnp.int32),
            pltpu.VMEM((chunk,), jnp.float32),
            pltpu.SemaphoreType.DMA,
            pltpu.SemaphoreType.DMA,
            pltpu.SemaphoreType.DMA,
            pltpu.SemaphoreType.DMA,
        ],
    )
    def k(opt_hbm, w_hbm, ol_hbm, out_hbm, w_v, ol_v, tab_v, idx_v, out_v,
          sem_in_lo, sem_in_hi, sem_tab, sem_out):
        wid = lax.axis_index("s") * n_cores + lax.axis_index("c")
        base = wid * chunk
        half = chunk // 2

        cp_in_lo = pltpu.async_copy(
            opt_hbm.at[pl.ds(base, half)], idx_v.at[pl.ds(0, half)], sem_in_lo
        )
        cp_in_hi = pltpu.async_copy(
            opt_hbm.at[pl.ds(base + half, half)],
            idx_v.at[pl.ds(half, half)],
            sem_in_hi,
        )
        cp_w = pltpu.async_copy(w_hbm, w_v, sem_tab)
        cp_ol = pltpu.async_copy(ol_hbm, ol_v, sem_tab)
        cp_w.wait()
        cp_ol.wait()

        for off in offs:
            ids = ol_v[pl.ds(off, _LANES)]
            vals = w_v[pl.ds(off, _LANES)]
            plsc.store_scatter(tab_v, [ids], vals)

        cp_in_lo.wait()

        @plsc.parallel_loop(0, half, step=_LANES, unroll=8)
        def body_lo(i):
            ids = idx_v[pl.ds(i, _LANES)]
            out_v[pl.ds(i, _LANES)] = plsc.load_gather(tab_v, [ids])

        cp_out = pltpu.async_copy(
            out_v.at[pl.ds(0, half)], out_hbm.at[pl.ds(base, half)], sem_out
        )
        cp_in_hi.wait()

        @plsc.parallel_loop(half, chunk, step=_LANES, unroll=8)
        def body_hi(i):
            ids = idx_v[pl.ds(i, _LANES)]
            out_v[pl.ds(i, _LANES)] = plsc.load_gather(tab_v, [ids])

        cp_out.wait()
        pltpu.sync_copy(
            out_v.at[pl.ds(half, half)], out_hbm.at[pl.ds(base + half, half)]
        )

    return k


def kernel(option, weights, option_list):
    m_total = option.size
    n_cores = 2
    n_workers = 16 * n_cores
    chunk = m_total // n_workers
    assert m_total % (n_workers * _LANES) == 0

    out = _sc_lookup(m_total, option_list.shape[0], n_cores, chunk)(
        option.reshape(-1), weights, option_list
    )
    return out.reshape(option.shape)


# speedup vs baseline: 1.0091x; 1.0091x over previous
import functools

import jax
import jax.numpy as jnp
from jax import lax
from jax.experimental import pallas as pl
from jax.experimental.pallas import tpu as pltpu
from jax.experimental.pallas import tpu_sc as plsc

_N_TABLE = 128
_LANES = 16


def _sc_lookup(m_total, n_opt, n_cores, chunk):
    mesh = plsc.VectorSubcoreMesh(
        core_axis_name="c", subcore_axis_name="s", num_cores=n_cores
    )

    offs = list(range(0, n_opt - _LANES + 1, _LANES))
    if n_opt % _LANES:
        offs.append(n_opt - _LANES)

    @functools.partial(
        pl.kernel,
        out_type=jax.ShapeDtypeStruct((m_total,), jnp.float32),
        mesh=mesh,
        compiler_params=pltpu.CompilerParams(needs_layout_passes=False),
        scratch_types=[
            pltpu.VMEM((n_opt,), jnp.float32),
            pltpu.VMEM((n_opt,), jnp.int32),
            pltpu.VMEM((_N_TABLE,), jnp.float32),
            pltpu.VMEM((chunk,), jnp.int32),
            pltpu.VMEM((chunk,), jnp.float32),
            pltpu.SemaphoreType.DMA,
            pltpu.SemaphoreType.DMA,
            pltpu.SemaphoreType.DMA,
            pltpu.SemaphoreType.DMA,
        ],
    )
    def k(opt_hbm, w_hbm, ol_hbm, out_hbm, w_v, ol_v, tab_v, idx_v, out_v,
          sem_in_lo, sem_in_hi, sem_tab, sem_out):
        wid = lax.axis_index("s") * n_cores + lax.axis_index("c")
        base = wid * chunk
        half = chunk // 2

        cp_in_lo = pltpu.async_copy(
            opt_hbm.at[pl.ds(base, half)], idx_v.at[pl.ds(0, half)], sem_in_lo
        )
        cp_in_hi = pltpu.async_copy(
            opt_hbm.at[pl.ds(base + half, half)],
            idx_v.at[pl.ds(half, half)],
            sem_in_hi,
        )
        cp_w = pltpu.async_copy(w_hbm, w_v, sem_tab)
        cp_ol = pltpu.async_copy(ol_hbm, ol_v, sem_tab)
        cp_w.wait()
        cp_ol.wait()

        for off in offs:
            ids = ol_v[pl.ds(off, _LANES)]
            vals = w_v[pl.ds(off, _LANES)]
            plsc.store_scatter(tab_v, [ids], vals)

        cp_in_lo.wait()

        @plsc.parallel_loop(0, half, step=_LANES, unroll=16)
        def body_lo(i):
            ids = idx_v[pl.ds(i, _LANES)]
            out_v[pl.ds(i, _LANES)] = plsc.load_gather(tab_v, [ids])

        cp_out = pltpu.async_copy(
            out_v.at[pl.ds(0, half)], out_hbm.at[pl.ds(base, half)], sem_out
        )
        cp_in_hi.wait()

        @plsc.parallel_loop(half, chunk, step=_LANES, unroll=16)
        def body_hi(i):
            ids = idx_v[pl.ds(i, _LANES)]
            out_v[pl.ds(i, _LANES)] = plsc.load_gather(tab_v, [ids])

        cp_out.wait()
        pltpu.sync_copy(
            out_v.at[pl.ds(half, half)], out_hbm.at[pl.ds(base + half, half)]
        )

    return k


def kernel(option, weights, option_list):
    m_total = option.size
    n_cores = 1
    n_workers = 16 * n_cores
    chunk = m_total // n_workers
    assert m_total % (n_workers * _LANES) == 0

    out = _sc_lookup(m_total, option_list.shape[0], n_cores, chunk)(
        option.reshape(-1), weights, option_list
    )
    return out.reshape(option.shape)


# speedup vs baseline: 1.0105x; 1.0014x over previous
import functools

import jax
import jax.numpy as jnp
from jax import lax
from jax.experimental import pallas as pl
from jax.experimental.pallas import tpu as pltpu
from jax.experimental.pallas import tpu_sc as plsc

_N_TABLE = 128
_LANES = 16
_N_SEG = 4


def _sc_lookup(m_total, n_opt, n_cores, chunk):
    mesh = plsc.VectorSubcoreMesh(
        core_axis_name="c", subcore_axis_name="s", num_cores=n_cores
    )

    offs = list(range(0, n_opt - _LANES + 1, _LANES))
    if n_opt % _LANES:
        offs.append(n_opt - _LANES)

    @functools.partial(
        pl.kernel,
        out_type=jax.ShapeDtypeStruct((m_total,), jnp.float32),
        mesh=mesh,
        compiler_params=pltpu.CompilerParams(needs_layout_passes=False),
        scratch_types=[
            pltpu.VMEM((n_opt,), jnp.float32),
            pltpu.VMEM((n_opt,), jnp.int32),
            pltpu.VMEM((_N_TABLE,), jnp.float32),
            pltpu.VMEM((chunk,), jnp.int32),
            pltpu.VMEM((chunk,), jnp.float32),
            pltpu.SemaphoreType.DMA,
        ] + [pltpu.SemaphoreType.DMA] * (2 * _N_SEG),
    )
    def k(opt_hbm, w_hbm, ol_hbm, out_hbm, w_v, ol_v, tab_v, idx_v, out_v,
          sem_tab, *sems):
        sems_in = sems[:_N_SEG]
        sems_out = sems[_N_SEG:]
        wid = lax.axis_index("s") * n_cores + lax.axis_index("c")
        base = wid * chunk
        seg = chunk // _N_SEG

        cps_in = [
            pltpu.async_copy(
                opt_hbm.at[pl.ds(base + q * seg, seg)],
                idx_v.at[pl.ds(q * seg, seg)],
                sems_in[q],
            )
            for q in range(_N_SEG)
        ]
        cp_w = pltpu.async_copy(w_hbm, w_v, sem_tab)
        cp_ol = pltpu.async_copy(ol_hbm, ol_v, sem_tab)
        cp_w.wait()
        cp_ol.wait()

        for off in offs:
            ids = ol_v[pl.ds(off, _LANES)]
            vals = w_v[pl.ds(off, _LANES)]
            plsc.store_scatter(tab_v, [ids], vals)

        cps_out = []
        for q in range(_N_SEG):
            cps_in[q].wait()

            @plsc.parallel_loop(q * seg, (q + 1) * seg, step=_LANES, unroll=8)
            def body(i):
                ids = idx_v[pl.ds(i, _LANES)]
                out_v[pl.ds(i, _LANES)] = plsc.load_gather(tab_v, [ids])

            cps_out.append(
                pltpu.async_copy(
                    out_v.at[pl.ds(q * seg, seg)],
                    out_hbm.at[pl.ds(base + q * seg, seg)],
                    sems_out[q],
                )
            )
        for cp in cps_out:
            cp.wait()

    return k


def kernel(option, weights, option_list):
    m_total = option.size
    n_cores = 1
    n_workers = 16 * n_cores
    chunk = m_total // n_workers
    assert m_total % (n_workers * _LANES) == 0

    out = _sc_lookup(m_total, option_list.shape[0], n_cores, chunk)(
        option.reshape(-1), weights, option_list
    )
    return out.reshape(option.shape)
